# baseline (speedup 1.0000x reference)
import functools

import jax
import jax.numpy as jnp
from jax import lax
from jax.experimental import pallas as pl
from jax.experimental.pallas import tpu as pltpu
from jax.experimental.pallas import tpu_sc as plsc

NUM_NODES = 4096
NUM_EDGES = 65536
L = 16
NC, NS = 2, 16
NW = NC * NS
E_HIST = NUM_EDGES // NS
E_GATH = NUM_EDGES // NW
NODES_PER_TILE = NUM_NODES // NS

_mesh = plsc.VectorSubcoreMesh(core_axis_name="c", subcore_axis_name="s")


def _rsqrt16(x):
    i = plsc.bitcast(x, jnp.int32)
    i = 0x5F3759DF - lax.shift_right_logical(i, 1)
    y = plsc.bitcast(i, jnp.float32)
    for _ in range(3):
        y = y * (1.5 - 0.5 * x * y * y)
    return jnp.where(x > 0.5, y, 0.0)


@functools.partial(
    pl.kernel,
    out_type=jax.ShapeDtypeStruct((NUM_EDGES,), jnp.float32),
    mesh=_mesh,
    compiler_params=pltpu.CompilerParams(
        needs_layout_passes=False,
        skip_device_barrier=True,
        disable_bounds_checks=True,
        disable_semaphore_checks=True,
    ),
    scratch_types=[
        pltpu.VMEM((E_HIST,), jnp.int32),
        pltpu.VMEM((NUM_NODES,), jnp.float32),
        pltpu.VMEM((NS, NODES_PER_TILE), jnp.float32),
        pltpu.VMEM((NODES_PER_TILE,), jnp.float32),
        pltpu.VMEM((NUM_NODES,), jnp.float32),
        pltpu.VMEM((E_GATH,), jnp.int32),
        pltpu.VMEM((E_GATH,), jnp.int32),
        pltpu.VMEM((E_GATH,), jnp.float32),
        pltpu.VMEM_SHARED((NS, NUM_NODES), jnp.float32),
        pltpu.VMEM_SHARED((NUM_NODES,), jnp.float32),
    ],
)
def _gcn_norm(ei_hbm, out_hbm, col_v, hist_v, part_v, dslice_v,
              table_v, rowi_v, coli_v, outb_v, stage_sh, dinv_sh):
    c = lax.axis_index("c")
    s = lax.axis_index("s")

    pltpu.sync_copy(ei_hbm.at[1, pl.ds(s * E_HIST, E_HIST)], col_v)

    @plsc.parallel_loop(0, NUM_NODES // L, unroll=8)
    def zero_body(i):
        hist_v[pl.ds(i * L, L)] = jnp.zeros((L,), jnp.float32)

    ones = jnp.ones((L,), jnp.float32)

    @plsc.parallel_loop(0, E_HIST // L, unroll=8)
    def hist_body(i):
        idx = col_v[pl.ds(i * L, L)]
        plsc.addupdate_scatter(hist_v, [idx], ones)

    pltpu.sync_copy(hist_v, stage_sh.at[s])
    plsc.subcore_barrier()
    base_n = s * NODES_PER_TILE
    pltpu.sync_copy(stage_sh.at[:, pl.ds(base_n, NODES_PER_TILE)], part_v)

    @plsc.parallel_loop(0, NODES_PER_TILE // L, unroll=2)
    def red_body(j):
        acc = jnp.zeros((L,), jnp.float32)
        for k in range(NS):
            acc = acc + part_v[k, pl.ds(j * L, L)]
        dslice_v[pl.ds(j * L, L)] = _rsqrt16(acc)

    pltpu.sync_copy(dslice_v, dinv_sh.at[pl.ds(base_n, NODES_PER_TILE)])
    plsc.subcore_barrier()
    pltpu.sync_copy(dinv_sh, table_v)

    w = s * NC + c
    base_e = w * E_GATH
    pltpu.sync_copy(ei_hbm.at[0, pl.ds(base_e, E_GATH)], rowi_v)
    pltpu.sync_copy(ei_hbm.at[1, pl.ds(base_e, E_GATH)], coli_v)

    @plsc.parallel_loop(0, E_GATH // L, unroll=8)
    def gath_body(i):
        r = rowi_v[pl.ds(i * L, L)]
        q = coli_v[pl.ds(i * L, L)]
        a = plsc.load_gather(table_v, [r])
        b = plsc.load_gather(table_v, [q])
        outb_v[pl.ds(i * L, L)] = a * b
    pltpu.sync_copy(outb_v, out_hbm.at[pl.ds(base_e, E_GATH)])


def kernel(edge_index):
    ei = edge_index
    if ei.dtype != jnp.int32:
        ei = ei.astype(jnp.int32)
    return _gcn_norm(ei)


# speedup vs baseline: 1.0673x; 1.0673x over previous
import functools

import jax
import jax.numpy as jnp
from jax import lax
from jax.experimental import pallas as pl
from jax.experimental.pallas import tpu as pltpu
from jax.experimental.pallas import tpu_sc as plsc

NUM_NODES = 4096
NUM_EDGES = 65536
L = 16
NC, NS = 2, 16
NW = NC * NS
E_HIST = NUM_EDGES // NS
E_GATH = NUM_EDGES // NW
NODES_PER_TILE = NUM_NODES // NS

_mesh = plsc.VectorSubcoreMesh(core_axis_name="c", subcore_axis_name="s")


def _rsqrt16(x):
    i = plsc.bitcast(x, jnp.int32)
    i = 0x5F3759DF - lax.shift_right_logical(i, 1)
    y = plsc.bitcast(i, jnp.float32)
    for _ in range(3):
        y = y * (1.5 - 0.5 * x * y * y)
    return jnp.where(x > 0.5, y, 0.0)


@functools.partial(
    pl.kernel,
    out_type=jax.ShapeDtypeStruct((NUM_EDGES,), jnp.float32),
    mesh=_mesh,
    compiler_params=pltpu.CompilerParams(
        needs_layout_passes=False,
        skip_device_barrier=True,
        disable_bounds_checks=True,
        disable_semaphore_checks=True,
    ),
    scratch_types=[
        pltpu.VMEM((E_HIST,), jnp.int32),
        pltpu.VMEM((NUM_NODES,), jnp.float32),
        pltpu.VMEM((NS, NODES_PER_TILE), jnp.float32),
        pltpu.VMEM((NODES_PER_TILE,), jnp.float32),
        pltpu.VMEM((NUM_NODES,), jnp.float32),
        pltpu.VMEM((E_GATH,), jnp.int32),
        pltpu.VMEM((E_GATH,), jnp.float32),
        pltpu.VMEM_SHARED((NS, NUM_NODES), jnp.float32),
        pltpu.VMEM_SHARED((NUM_NODES,), jnp.float32),
        pltpu.SemaphoreType.DMA,
        pltpu.SemaphoreType.DMA,
    ],
)
def _gcn_norm(ei_hbm, out_hbm, col_v, hist_v, part_v, dslice_v,
              table_v, rowi_v, outb_v, stage_sh, dinv_sh, sem_c, sem_r):
    c = lax.axis_index("c")
    s = lax.axis_index("s")
    w = s * NC + c
    base_e = w * E_GATH

    cp_col = pltpu.async_copy(ei_hbm.at[1, pl.ds(s * E_HIST, E_HIST)], col_v,
                              sem_c)
    cp_row = pltpu.async_copy(ei_hbm.at[0, pl.ds(base_e, E_GATH)], rowi_v,
                              sem_r)

    @plsc.parallel_loop(0, NUM_NODES // L, unroll=8)
    def zero_body(i):
        hist_v[pl.ds(i * L, L)] = jnp.zeros((L,), jnp.float32)

    cp_col.wait()
    ones = jnp.ones((L,), jnp.float32)

    @plsc.parallel_loop(0, E_HIST // L, unroll=8)
    def hist_body(i):
        idx = col_v[pl.ds(i * L, L)]
        plsc.addupdate_scatter(hist_v, [idx], ones)

    pltpu.sync_copy(hist_v, stage_sh.at[s])
    plsc.subcore_barrier()
    base_n = s * NODES_PER_TILE
    pltpu.sync_copy(stage_sh.at[:, pl.ds(base_n, NODES_PER_TILE)], part_v)

    @plsc.parallel_loop(0, NODES_PER_TILE // L, unroll=2)
    def red_body(j):
        acc = jnp.zeros((L,), jnp.float32)
        for k in range(NS):
            acc = acc + part_v[k, pl.ds(j * L, L)]
        dslice_v[pl.ds(j * L, L)] = _rsqrt16(acc)

    pltpu.sync_copy(dslice_v, dinv_sh.at[pl.ds(base_n, NODES_PER_TILE)])
    plsc.subcore_barrier()
    pltpu.sync_copy(dinv_sh, table_v)

    cp_row.wait()
    col_off = c * E_GATH

    @plsc.parallel_loop(0, E_GATH // L, unroll=8)
    def gath_body(i):
        r = rowi_v[pl.ds(i * L, L)]
        q = col_v[pl.ds(col_off + i * L, L)]
        a = plsc.load_gather(table_v, [r])
        b = plsc.load_gather(table_v, [q])
        outb_v[pl.ds(i * L, L)] = a * b
    pltpu.sync_copy(outb_v, out_hbm.at[pl.ds(base_e, E_GATH)])


def kernel(edge_index):
    ei = edge_index
    if ei.dtype != jnp.int32:
        ei = ei.astype(jnp.int32)
    return _gcn_norm(ei)
